# baseline (speedup 1.0000x reference)
import functools

import jax
import jax.numpy as jnp
from jax import lax
from jax.experimental import pallas as pl
from jax.experimental.pallas import tpu as pltpu
from jax.experimental.pallas import tpu_sc as plsc

BATCH = 16384
FIELDS = 26
FH = FIELDS // 2
D = 64
N_ROWS = BATCH * FIELDS
NH_ROWS = BATCH * FH
HALF_B = BATCH // 2
V = 1000000

SB = 8192
NSB_MAIN = V // SB
V_MAIN = NSB_MAIN * SB
TAIL = V - V_MAIN
PACK_ROWS = SB // 2
TAIL_PAIRS = TAIL // 2


def _pack_body(x_ref, tail_ref, o_ref):
    k = pl.program_id(0)

    @pl.when(k < NSB_MAIN)
    def _():
        x = x_ref[...]
        o_ref[...] = jnp.concatenate(
            [x[:, :PACK_ROWS].T, x[:, PACK_ROWS:].T], axis=1
        )

    @pl.when(k == NSB_MAIN)
    def _():
        t = tail_ref[...]
        o_ref[pl.ds(0, TAIL_PAIRS), :] = jnp.concatenate(
            [t[:, :TAIL_PAIRS].T, t[:, TAIL_PAIRS:].T], axis=1
        )


def _pack_table(w_emb_t, tail):
    return pl.pallas_call(
        _pack_body,
        grid=(NSB_MAIN + 1,),
        in_specs=[
            pl.BlockSpec((D, SB), lambda k: (0, jnp.minimum(k, NSB_MAIN - 1))),
            pl.BlockSpec((D, TAIL), lambda k: (0, 0)),
        ],
        out_specs=pl.BlockSpec((PACK_ROWS, 2 * D), lambda k: (k, 0)),
        out_shape=jax.ShapeDtypeStruct((V // 2, 2 * D), jnp.float32),
    )(w_emb_t, tail)


NC = 2
NS = 16
NW = NC * NS
BLOCK_PAIRS = 256
BLOCK_ROWS = 2 * BLOCK_PAIRS
NBLK_H = NH_ROWS // BLOCK_ROWS
BLK_PER_W = NBLK_H // NW
BLK_PER_F = BATCH // (2 * BLOCK_PAIRS)
CH = 128
NCH = BLOCK_ROWS // CH
IDS_R = BATCH // CH
IDR_B = BLOCK_PAIRS // CH

_sc_mesh = plsc.VectorSubcoreMesh(
    core_axis_name="c", subcore_axis_name="s", num_cores=NC, num_subcores=NS
)


def _make_sc_gather(fb):
    @functools.partial(
        pl.kernel,
        mesh=_sc_mesh,
        out_type=jax.ShapeDtypeStruct((NH_ROWS, D), jnp.float32),
        scratch_types=[
            pltpu.VMEM((IDR_B, CH), jnp.int32),
            pltpu.VMEM((IDR_B, CH), jnp.int32),
            pltpu.VMEM((BLOCK_ROWS,), jnp.int32),
            pltpu.VMEM((BLOCK_ROWS, D), jnp.float32),
            pltpu.SemaphoreType.DMA,
        ],
        compiler_params=pltpu.CompilerParams(
            use_tc_tiling_on_sc=False, needs_layout_passes=False
        ),
        name=f"sc_gather_{fb}",
    )
    def sc_gather(ids_hbm, table_hbm, out_hbm, lo_v, hi_v, ilv_v, rows_v, gsem):
        wid = lax.axis_index("s") * NC + lax.axis_index("c")
        iota = lax.iota(jnp.int32, 16)

        def block_body(k, carry):
            blk = wid + NW * k
            f = fb + blk // BLK_PER_F
            row0 = (blk % BLK_PER_F) * IDR_B
            pltpu.sync_copy(ids_hbm.at[f, pl.ds(row0, IDR_B)], lo_v)
            pltpu.sync_copy(ids_hbm.at[f, pl.ds(IDS_R // 2 + row0, IDR_B)], hi_v)
            for t in range(BLOCK_PAIRS // 16):
                pos = 2 * (16 * t + iota)
                plsc.store_scatter(
                    ilv_v, [pos], lo_v[t // 8, pl.ds((t % 8) * 16, 16)]
                )
                plsc.store_scatter(
                    ilv_v, [pos + 1], hi_v[t // 8, pl.ds((t % 8) * 16, 16)]
                )
            copies = [
                pltpu.async_copy(
                    table_hbm.at[ilv_v.at[pl.ds(j * CH, CH)]],
                    rows_v.at[pl.ds(j * CH, CH)],
                    gsem,
                )
                for j in range(NCH)
            ]
            for c in copies:
                c.wait()
            pltpu.sync_copy(
                rows_v, out_hbm.at[pl.ds(blk * BLOCK_ROWS, BLOCK_ROWS)]
            )
            return carry

        lax.fori_loop(0, BLK_PER_W, block_body, 0)

    return sc_gather


_sc_gather_0 = _make_sc_gather(0)
_sc_gather_1 = _make_sc_gather(FH)

_SQRT_HALF = 0.7071067811865476


def _mlp_body(x_ref, w1l_ref, w1r_ref, b1_ref, w2_ref, b2c_ref, o_ref):
    x = x_ref[...]
    b1 = b1_ref[...]
    b2c = b2c_ref[...]
    w2 = w2_ref[...]
    for half, w1_ref in ((0, w1l_ref), (1, w1r_ref)):
        h = jnp.dot(x, w1_ref[...], preferred_element_type=jnp.float32) + b1
        h = h * 0.5 * (1.0 + lax.erf(h * _SQRT_HALF))
        y = lax.dot_general(
            w2, h, (((1,), (1,)), ((), ())), preferred_element_type=jnp.float32
        )
        o_ref[0, :, pl.ds(half * HALF_B, HALF_B)] = y + b2c


def _mlp_body_alias(y_ref, *rest):
    del y_ref
    _mlp_body(*rest)


_MLP_SPECS = [
    pl.BlockSpec((HALF_B, 2 * D), lambda f: (f, 0)),
    pl.BlockSpec((2 * D, D), lambda f: (0, 0)),
    pl.BlockSpec((2 * D, D), lambda f: (0, 0)),
    pl.BlockSpec((1, D), lambda f: (0, 0)),
    pl.BlockSpec((D, D), lambda f: (0, 0)),
    pl.BlockSpec((D, 1), lambda f: (0, 0)),
]


def _mlp_first(staging128, w1l, w1r, b1, w2, b2c):
    return pl.pallas_call(
        _mlp_body,
        grid=(FH,),
        in_specs=_MLP_SPECS,
        out_specs=pl.BlockSpec((1, D, BATCH), lambda f: (f, 0, 0)),
        out_shape=jax.ShapeDtypeStruct((FIELDS, D, BATCH), jnp.float32),
        name="mlp_0",
    )(staging128, w1l, w1r, b1, w2, b2c)


def _mlp_second(y_prev, staging128, w1l, w1r, b1, w2, b2c):
    return pl.pallas_call(
        _mlp_body_alias,
        grid=(FH,),
        in_specs=[pl.BlockSpec(memory_space=pl.ANY)] + _MLP_SPECS,
        out_specs=pl.BlockSpec((1, D, BATCH), lambda f: (f + FH, 0, 0)),
        out_shape=jax.ShapeDtypeStruct((FIELDS, D, BATCH), jnp.float32),
        input_output_aliases={0: 0},
        name="mlp_1",
    )(y_prev, staging128, w1l, w1r, b1, w2, b2c)


def kernel(action_ids, W_emb, W1, b1, W2, b2):
    wt = W_emb.T
    tlin = _pack_table(wt, wt[:, V_MAIN:]).reshape(V, D)
    j = action_ids.T
    o = j % SB
    jj_main = 2 * (PACK_ROWS * (j // SB) + o % PACK_ROWS) + o // PACK_ROWS
    ot = j - V_MAIN
    jj_tail = 2 * (V_MAIN // 2 + ot % TAIL_PAIRS) + ot // TAIL_PAIRS
    jj = jnp.where(j < V_MAIN, jj_main, jj_tail)
    ids3 = jj.reshape(FIELDS, IDS_R, CH)
    w1t = W1.T
    zeros = jnp.zeros((D, D), jnp.float32)
    w1l = jnp.concatenate([w1t, zeros], axis=0)
    w1r = jnp.concatenate([zeros, w1t], axis=0)
    b1r = b1.reshape(1, D)
    b2c = b2.reshape(D, 1)
    staging_0 = _sc_gather_0(ids3, tlin)
    y0 = _mlp_first(
        staging_0.reshape(NH_ROWS // 2, 2 * D), w1l, w1r, b1r, W2, b2c
    )
    staging_1 = _sc_gather_1(ids3, tlin)
    y1 = _mlp_second(
        y0, staging_1.reshape(NH_ROWS // 2, 2 * D), w1l, w1r, b1r, W2, b2c
    )
    return y1.transpose(2, 0, 1)
